# baseline (speedup 1.0000x reference)
import functools

import jax
import jax.numpy as jnp
from jax import lax
from jax.experimental import pallas as pl
from jax.experimental.pallas import tpu as pltpu
from jax.experimental.pallas import tpu_sc as plsc

NC = 2
NS = 16
L = 16
K = 128
NW = NC * NS


DW = 128


def _make_degrees(npad, ch_e, nset):
    rpt = npad // NS
    zr = 8
    mesh = plsc.VectorSubcoreMesh(core_axis_name="c", subcore_axis_name="s")

    def body(idx_hbm, out_hbm, idxv, ones, zbuf, sdeg):
        cid = lax.axis_index("c")
        sid = lax.axis_index("s")

        def fill(r, carry):
            for k in range(DW // L):
                ones[r, pl.ds(k * L, L)] = jnp.full((L,), 1.0, jnp.float32)
            return carry
        lax.fori_loop(0, K, fill, 0)

        def zfill(r, carry):
            for k in range(DW // L):
                zbuf[r, pl.ds(k * L, L)] = jnp.zeros((L,), jnp.float32)
            return carry
        lax.fori_loop(0, zr, zfill, 0)

        def zcopy(t, carry):
            pltpu.sync_copy(zbuf, sdeg.at[pl.ds(sid * rpt + t * zr, zr)])
            return carry
        lax.fori_loop(0, rpt // zr, zcopy, 0)

        for a in range(nset):
            pltpu.sync_copy(idx_hbm.at[a, cid, sid], idxv)
            plsc.subcore_barrier()

            def acc(j, carry):
                pltpu.sync_copy(ones, sdeg.at[idxv.at[j]], add=True)
                return carry
            lax.fori_loop(0, ch_e, acc, 0)

            plsc.subcore_barrier()
            pltpu.sync_copy(sdeg.at[pl.ds(sid * rpt, rpt)],
                            out_hbm.at[cid, pl.ds(a * npad + sid * rpt, rpt)])
            if a < nset - 1:
                lax.fori_loop(0, rpt // zr, zcopy, 0)
                plsc.subcore_barrier()

    return functools.partial(
        pl.kernel,
        out_type=jax.ShapeDtypeStruct((NC, nset * npad, DW), jnp.float32),
        mesh=mesh,
        scratch_types=[
            pltpu.VMEM((ch_e, K), jnp.int32),
            pltpu.VMEM((K, DW), jnp.float32),
            pltpu.VMEM((zr, DW), jnp.float32),
            pltpu.VMEM_SHARED((npad, DW), jnp.float32),
        ],
    )(body)


NB = 2
NSEG = 2


def _make_propagate(npad, ch_e, d):
    rpt = npad // NS
    zr = 8
    seg = ch_e // NSEG
    mesh = plsc.VectorSubcoreMesh(core_axis_name="c", subcore_axis_name="s")

    def body(msg_hbm, src_hbm, dst_hbm, out_hbm,
             srcv, dstv, bufs, zbuf, s0, s1, aggsp):
        cid = lax.axis_index("c")
        sid = lax.axis_index("s")
        sems = [s0, s1]

        def zero_row(r, carry):
            for k in range(d // L):
                zbuf[r, pl.ds(k * L, L)] = jnp.zeros((L,), jnp.float32)
            return carry
        lax.fori_loop(0, zr, zero_row, 0)

        def zcopy(t, carry):
            pltpu.sync_copy(zbuf, aggsp.at[pl.ds(sid * rpt + t * zr, zr)])
            return carry
        lax.fori_loop(0, rpt // zr, zcopy, 0)

        for sg in range(NSEG):
            pltpu.sync_copy(src_hbm.at[cid, sid, pl.ds(sg * seg, seg)], srcv)
            pltpu.sync_copy(dst_hbm.at[cid, sid, pl.ds(sg * seg, seg)], dstv)
            if sg == 0:
                plsc.subcore_barrier()

            for b in range(NB):
                pltpu.async_copy(msg_hbm.at[srcv.at[b]],
                                 bufs.at[pl.ds(b * K, K)], sems[b])

            def group(g, carry):
                for b in range(NB):
                    j = g * NB + b
                    slot = bufs.at[pl.ds(b * K, K)]
                    pltpu.make_async_copy(msg_hbm.at[srcv.at[j]],
                                          slot, sems[b]).wait()
                    pltpu.sync_copy(slot, aggsp.at[dstv.at[j]], add=True)

                    @pl.when(j + NB < seg)
                    def _():
                        pltpu.async_copy(msg_hbm.at[srcv.at[j + NB]],
                                         slot, sems[b])
                return carry
            lax.fori_loop(0, seg // NB, group, 0)

        plsc.subcore_barrier()
        pltpu.sync_copy(aggsp.at[pl.ds(sid * rpt, rpt)],
                        out_hbm.at[cid, pl.ds(sid * rpt, rpt)])

    return functools.partial(
        pl.kernel,
        out_type=jax.ShapeDtypeStruct((NC, npad, d), jnp.float32),
        mesh=mesh,
        scratch_types=[
            pltpu.VMEM((ch_e // NSEG, K), jnp.int32),
            pltpu.VMEM((ch_e // NSEG, K), jnp.int32),
            pltpu.VMEM((NB * K, d), jnp.float32),
            pltpu.VMEM((zr, d), jnp.float32),
            pltpu.SemaphoreType.DMA,
            pltpu.SemaphoreType.DMA,
            pltpu.VMEM_SHARED((npad, d), jnp.float32),
        ],
    )(body)


def _tc_msg1(degp, x_pad, din):
    npad = x_pad.shape[0]

    def body(deg_ref, x_ref, msg_ref):
        deg = deg_ref[0] + deg_ref[1]
        ns1 = lax.rsqrt(jnp.maximum(deg[:, 0:1], 1.0))
        msg_ref[...] = x_ref[...] * ns1

    return pl.pallas_call(
        body,
        grid=(npad // K,),
        in_specs=[
            pl.BlockSpec((NC, K, DW), lambda j: (0, j, 0)),
            pl.BlockSpec((K, din), lambda j: (j, 0)),
        ],
        out_specs=pl.BlockSpec((K, din), lambda j: (j, 0)),
        out_shape=jax.ShapeDtypeStruct((npad, din), jnp.float32),
    )(degp, x_pad)


def _tc_layer1_msg2(aggp, degp, w1, b1r, chn):
    npad = aggp.shape[1]
    din = w1.shape[0]
    dh = w1.shape[1]

    def body(aggp_ref, deg1_ref, deg2_ref, w1_ref, b1_ref, msg2_ref):
        agg = aggp_ref[0] + aggp_ref[1]
        deg1 = deg1_ref[0] + deg1_ref[1]
        nd1 = lax.rsqrt(jnp.maximum(deg1[:, 0:1], 1.0))
        h = jnp.dot(agg * nd1, w1_ref[...],
                    preferred_element_type=jnp.float32) + b1_ref[0:1]
        h = jnp.maximum(h, 0.0)
        deg2 = deg2_ref[0] + deg2_ref[1]
        ns2 = lax.rsqrt(jnp.maximum(deg2[:, 0:1], 1.0))
        msg2_ref[...] = h * ns2

    return pl.pallas_call(
        body,
        grid=(npad // K,),
        in_specs=[
            pl.BlockSpec((NC, K, din), lambda j: (0, j, 0)),
            pl.BlockSpec((NC, K, DW), lambda j: (0, chn + j, 0)),
            pl.BlockSpec((NC, K, DW), lambda j: (0, 2 * chn + j, 0)),
            pl.BlockSpec((din, dh), lambda j: (0, 0)),
            pl.BlockSpec((8, dh), lambda j: (0, 0)),
        ],
        out_specs=pl.BlockSpec((K, dh), lambda j: (j, 0)),
        out_shape=jax.ShapeDtypeStruct((npad, dh), jnp.float32),
    )(aggp, degp, degp, w1, b1r)


def _tc_final(aggp2, degp, w2, b2r, chn):
    npad = aggp2.shape[1]
    dh = w2.shape[0]
    dout = w2.shape[1]

    def body(aggp_ref, deg_ref, w2_ref, b2_ref, out_ref):
        agg = aggp_ref[0] + aggp_ref[1]
        deg = deg_ref[0] + deg_ref[1]
        nd2 = lax.rsqrt(jnp.maximum(deg[:, 0:1], 1.0))
        o = jnp.dot(agg * nd2, w2_ref[...],
                    preferred_element_type=jnp.float32) + b2_ref[0:1]
        o = jnp.maximum(o, 0.0)
        m = jnp.max(o, axis=1, keepdims=True)
        ex = jnp.exp(o - m)
        s = jnp.sum(ex, axis=1, keepdims=True)
        out_ref[...] = (o - m) - jnp.log(s)

    return pl.pallas_call(
        body,
        grid=(npad // K,),
        in_specs=[
            pl.BlockSpec((NC, K, dh), lambda j: (0, j, 0)),
            pl.BlockSpec((NC, K, DW), lambda j: (0, 3 * chn + j, 0)),
            pl.BlockSpec((dh, dout), lambda j: (0, 0)),
            pl.BlockSpec((8, dout), lambda j: (0, 0)),
        ],
        out_specs=pl.BlockSpec((K, dout), lambda j: (j, 0)),
        out_shape=jax.ShapeDtypeStruct((npad, dout), jnp.float32),
    )(aggp2, degp, w2, b2r)


def kernel(in_feat, edge_index1, edge_index2, W1, b1, W2, b2):
    n, din = in_feat.shape
    dh = W1.shape[1]
    dout = W2.shape[1]
    e = edge_index1.shape[1]

    chn = -(-(n + 1) // K)
    npad = chn * K
    ch_e = -(-(-(-e // (NW * K))) // (NSEG * NB)) * (NSEG * NB)
    ept = ch_e * K
    pad_e = ept * NW - e

    def prep(eidx):
        padv = jnp.full((pad_e,), n, jnp.int32)
        s = jnp.concatenate([eidx[0], padv]).reshape(NC, NS, ch_e, K)
        d = jnp.concatenate([eidx[1], padv]).reshape(NC, NS, ch_e, K)
        return s, d

    s1, d1 = prep(edge_index1)
    s2, d2 = prep(edge_index2)
    degidx = jnp.stack([s1, d1, s2, d2])
    x_pad = jnp.pad(in_feat, ((0, npad - n), (0, 0)))

    degp = _make_degrees(npad, ch_e, 4)(degidx)

    msg1 = _tc_msg1(degp, x_pad, din)

    aggp1 = _make_propagate(npad, ch_e, din)(msg1, s1, d1)

    b1r = jnp.broadcast_to(b1[None, :], (8, dh))
    msg2 = _tc_layer1_msg2(aggp1, degp, W1, b1r, chn)

    aggp2 = _make_propagate(npad, ch_e, dh)(msg2, s2, d2)

    b2r = jnp.broadcast_to(b2[None, :], (8, dout))
    out = _tc_final(aggp2, degp, W2, b2r, chn)
    return out[:n]


# speedup vs baseline: 1.4219x; 1.4219x over previous
import functools

import jax
import jax.numpy as jnp
from jax import lax
from jax.experimental import pallas as pl
from jax.experimental.pallas import tpu as pltpu
from jax.experimental.pallas import tpu_sc as plsc

NC = 2
NS = 16
L = 16
K = 128
NW = NC * NS


DW = 128


def _make_degrees(npad, ch_e, nset):
    rpt = npad // NS
    zr = 8
    mesh = plsc.VectorSubcoreMesh(core_axis_name="c", subcore_axis_name="s")

    def body(idx_hbm, out_hbm, idxv, ones, zbuf, sdeg):
        cid = lax.axis_index("c")
        sid = lax.axis_index("s")

        def fill(r, carry):
            for k in range(DW // L):
                ones[r, pl.ds(k * L, L)] = jnp.full((L,), 1.0, jnp.float32)
            return carry
        lax.fori_loop(0, K, fill, 0)

        def zfill(r, carry):
            for k in range(DW // L):
                zbuf[r, pl.ds(k * L, L)] = jnp.zeros((L,), jnp.float32)
            return carry
        lax.fori_loop(0, zr, zfill, 0)

        def zcopy(t, carry):
            pltpu.sync_copy(zbuf, sdeg.at[pl.ds(sid * rpt + t * zr, zr)])
            return carry
        lax.fori_loop(0, rpt // zr, zcopy, 0)

        for a in range(nset):
            pltpu.sync_copy(idx_hbm.at[a, cid, sid], idxv)
            plsc.subcore_barrier()

            def acc(j, carry):
                pltpu.sync_copy(ones, sdeg.at[idxv.at[j]], add=True)
                return carry
            lax.fori_loop(0, ch_e, acc, 0)

            plsc.subcore_barrier()
            pltpu.sync_copy(sdeg.at[pl.ds(sid * rpt, rpt)],
                            out_hbm.at[cid, pl.ds(a * npad + sid * rpt, rpt)])
            if a < nset - 1:
                lax.fori_loop(0, rpt // zr, zcopy, 0)
                plsc.subcore_barrier()

    return functools.partial(
        pl.kernel,
        out_type=jax.ShapeDtypeStruct((NC, nset * npad, DW), jnp.float32),
        mesh=mesh,
        scratch_types=[
            pltpu.VMEM((ch_e, K), jnp.int32),
            pltpu.VMEM((K, DW), jnp.float32),
            pltpu.VMEM((zr, DW), jnp.float32),
            pltpu.VMEM_SHARED((npad, DW), jnp.float32),
        ],
    )(body)


def _make_propagate(npad, ch_e, d):
    rpt = npad // NS
    zr = 8
    mesh = plsc.VectorSubcoreMesh(core_axis_name="c", subcore_axis_name="s")

    def body(msg_hbm, src_hbm, dst_hbm, out_hbm,
             srcv, dstv, buf, zbuf, sem, aggsp):
        cid = lax.axis_index("c")
        sid = lax.axis_index("s")

        def zero_row(r, carry):
            for k in range(d // L):
                zbuf[r, pl.ds(k * L, L)] = jnp.zeros((L,), jnp.float32)
            return carry
        lax.fori_loop(0, zr, zero_row, 0)

        def zcopy(t, carry):
            pltpu.sync_copy(zbuf, aggsp.at[pl.ds(sid * rpt + t * zr, zr)])
            return carry
        lax.fori_loop(0, rpt // zr, zcopy, 0)

        pltpu.sync_copy(src_hbm.at[cid, sid], srcv)
        pltpu.sync_copy(dst_hbm.at[cid, sid], dstv)
        plsc.subcore_barrier()

        def chunk(j, carry):
            pltpu.async_copy(msg_hbm.at[srcv.at[j]], buf, sem).wait()
            pltpu.sync_copy(buf, aggsp.at[dstv.at[j]], add=True)
            return carry
        lax.fori_loop(0, ch_e, chunk, 0)

        plsc.subcore_barrier()
        pltpu.sync_copy(aggsp.at[pl.ds(sid * rpt, rpt)],
                        out_hbm.at[cid, pl.ds(sid * rpt, rpt)])

    return functools.partial(
        pl.kernel,
        out_type=jax.ShapeDtypeStruct((NC, npad, d), jnp.float32),
        mesh=mesh,
        scratch_types=[
            pltpu.VMEM((ch_e, K), jnp.int32),
            pltpu.VMEM((ch_e, K), jnp.int32),
            pltpu.VMEM((K, d), jnp.float32),
            pltpu.VMEM((zr, d), jnp.float32),
            pltpu.SemaphoreType.DMA,
            pltpu.VMEM_SHARED((npad, d), jnp.float32),
        ],
    )(body)


def _tc_msg1(degp, x_pad, din):
    npad = x_pad.shape[0]

    def body(deg_ref, x_ref, msg_ref):
        deg = deg_ref[0] + deg_ref[1]
        ns1 = lax.rsqrt(jnp.maximum(deg[:, 0:1], 1.0))
        msg_ref[...] = x_ref[...] * ns1

    return pl.pallas_call(
        body,
        grid=(npad // K,),
        in_specs=[
            pl.BlockSpec((NC, K, DW), lambda j: (0, j, 0)),
            pl.BlockSpec((K, din), lambda j: (j, 0)),
        ],
        out_specs=pl.BlockSpec((K, din), lambda j: (j, 0)),
        out_shape=jax.ShapeDtypeStruct((npad, din), jnp.float32),
    )(degp, x_pad)


def _tc_layer1_msg2(aggp, degp1, degp2, w1, b1r, chn):
    npad = aggp.shape[1]
    din = w1.shape[0]
    dh = w1.shape[1]

    def body(aggp_ref, deg1_ref, deg2_ref, w1_ref, b1_ref, msg2_ref):
        agg = aggp_ref[0] + aggp_ref[1]
        deg1 = deg1_ref[0] + deg1_ref[1]
        nd1 = lax.rsqrt(jnp.maximum(deg1[:, 0:1], 1.0))
        h = jnp.dot(agg * nd1, w1_ref[...],
                    preferred_element_type=jnp.float32) + b1_ref[0:1]
        h = jnp.maximum(h, 0.0)
        deg2 = deg2_ref[0] + deg2_ref[1]
        ns2 = lax.rsqrt(jnp.maximum(deg2[:, 0:1], 1.0))
        msg2_ref[...] = h * ns2

    return pl.pallas_call(
        body,
        grid=(npad // K,),
        in_specs=[
            pl.BlockSpec((NC, K, din), lambda j: (0, j, 0)),
            pl.BlockSpec((NC, K, DW), lambda j: (0, chn + j, 0)),
            pl.BlockSpec((NC, K, DW), lambda j: (0, j, 0)),
            pl.BlockSpec((din, dh), lambda j: (0, 0)),
            pl.BlockSpec((8, dh), lambda j: (0, 0)),
        ],
        out_specs=pl.BlockSpec((K, dh), lambda j: (j, 0)),
        out_shape=jax.ShapeDtypeStruct((npad, dh), jnp.float32),
    )(aggp, degp1, degp2, w1, b1r)


def _tc_final(aggp2, degp, w2, b2r, chn):
    npad = aggp2.shape[1]
    dh = w2.shape[0]
    dout = w2.shape[1]

    def body(aggp_ref, deg_ref, w2_ref, b2_ref, out_ref):
        agg = aggp_ref[0] + aggp_ref[1]
        deg = deg_ref[0] + deg_ref[1]
        nd2 = lax.rsqrt(jnp.maximum(deg[:, 0:1], 1.0))
        o = jnp.dot(agg * nd2, w2_ref[...],
                    preferred_element_type=jnp.float32) + b2_ref[0:1]
        o = jnp.maximum(o, 0.0)
        m = jnp.max(o, axis=1, keepdims=True)
        ex = jnp.exp(o - m)
        s = jnp.sum(ex, axis=1, keepdims=True)
        out_ref[...] = (o - m) - jnp.log(s)

    return pl.pallas_call(
        body,
        grid=(npad // K,),
        in_specs=[
            pl.BlockSpec((NC, K, dh), lambda j: (0, j, 0)),
            pl.BlockSpec((NC, K, DW), lambda j: (0, chn + j, 0)),
            pl.BlockSpec((dh, dout), lambda j: (0, 0)),
            pl.BlockSpec((8, dout), lambda j: (0, 0)),
        ],
        out_specs=pl.BlockSpec((K, dout), lambda j: (j, 0)),
        out_shape=jax.ShapeDtypeStruct((npad, dout), jnp.float32),
    )(aggp2, degp, w2, b2r)


def kernel(in_feat, edge_index1, edge_index2, W1, b1, W2, b2):
    n, din = in_feat.shape
    dh = W1.shape[1]
    dout = W2.shape[1]
    e = edge_index1.shape[1]

    chn = -(-(n + 1) // K)
    npad = chn * K
    ept = -(-e // (NW * K)) * K
    ch_e = ept // K
    pad_e = ept * NW - e

    def prep(eidx):
        padv = jnp.full((pad_e,), n, jnp.int32)
        s = jnp.concatenate([eidx[0], padv]).reshape(NC, NS, ch_e, K)
        d = jnp.concatenate([eidx[1], padv]).reshape(NC, NS, ch_e, K)
        return s, d

    s1, d1 = prep(edge_index1)
    s2, d2 = prep(edge_index2)
    x_pad = jnp.pad(in_feat, ((0, npad - n), (0, 0)))

    mkdeg = _make_degrees(npad, ch_e, 2)
    degp1 = mkdeg(jnp.stack([s1, d1]))
    degp2 = mkdeg(jnp.stack([s2, d2]))

    msg1 = _tc_msg1(degp1, x_pad, din)

    aggp1 = _make_propagate(npad, ch_e, din)(msg1, s1, d1)

    b1r = jnp.broadcast_to(b1[None, :], (8, dh))
    msg2 = _tc_layer1_msg2(aggp1, degp1, degp2, W1, b1r, chn)

    aggp2 = _make_propagate(npad, ch_e, dh)(msg2, s2, d2)

    b2r = jnp.broadcast_to(b2[None, :], (8, dout))
    out = _tc_final(aggp2, degp2, W2, b2r, chn)
    return out[:n]
